# baseline (speedup 1.0000x reference)
import functools

import jax
import jax.numpy as jnp
import numpy as np
from jax.experimental import pallas as pl
from jax.experimental.pallas import tpu as pltpu

XMIN = 0.1
XMAX = 2.0

_HI = jax.lax.Precision.HIGHEST

_PIO2_H1 = np.float32(1.5703125)
_PIO2_H2 = np.float32(np.pi / 2 - 1.5703125)
_PIO2_H3 = np.float32(np.pi / 2 - 1.5703125 - float(np.float32(np.pi / 2 - 1.5703125)))
_INV_PIO2 = np.float32(2.0 / np.pi)
_S1 = np.float32(-1.6666654611e-1)
_S2 = np.float32(8.3321608736e-3)
_S3 = np.float32(-1.9515295891e-4)
_C1 = np.float32(4.166664568298827e-2)
_C2 = np.float32(-1.388731625493765e-3)
_C3 = np.float32(2.443315711809948e-5)


def _fast_sin(ang):
    nf = jnp.floor(ang * _INV_PIO2 + 0.5)
    r = ang - nf * _PIO2_H1
    r = r - nf * _PIO2_H2
    r = r - nf * _PIO2_H3
    ni = nf.astype(jnp.int32)
    r2 = r * r
    sp = ((_S3 * r2 + _S2) * r2 + _S1) * (r2 * r) + r
    cp = ((_C3 * r2 + _C2) * r2 + _C1) * (r2 * r2) + (1.0 - 0.5 * r2)
    res = jnp.where((ni & 1) == 0, sp, cp)
    return jnp.where((ni & 2) == 0, res, -res)


def _onehot(base, lo_ref, hi_ref, T, B):
    idx = jax.lax.broadcasted_iota(jnp.int32, (T, B), 0) + base
    return jnp.where((idx >= lo_ref[...]) & (idx < hi_ref[...]), 1.0, 0.0)


def _pass_a(flat_ref, s2_ref, inv_ref, lo_ref, hi_ref, w_ref, b_ref,
            ytop_ref, segacc_ref, *, T, E, B):
    i = pl.program_id(0)
    onehot = _onehot(i * T, lo_ref, hi_ref, T, B)
    x = flat_ref[...]
    x1 = x.astype(jnp.bfloat16)
    x2 = (x - x1.astype(jnp.float32)).astype(jnp.bfloat16)
    xb = jax.lax.dot_general(jnp.concatenate([x1, x2], axis=1), s2_ref[...],
                             (((1,), (0,)), ((), ())),
                             preferred_element_type=jnp.float32)
    ang = xb * inv_ref[...]
    nf = jnp.floor(ang * _INV_PIO2 + 0.5)
    r = ang - nf * _PIO2_H1
    r = r - nf * _PIO2_H2
    r = r - nf * _PIO2_H3
    ni = nf.astype(jnp.int32)
    r2 = r * r
    sp = ((_S3 * r2 + _S2) * r2 + _S1) * (r2 * r) + r
    cp = ((_C3 * r2 + _C2) * r2 + _C1) * (r2 * r2) + (1.0 - 0.5 * r2)
    odd = (ni & 1) == 0
    sinv = jnp.where(odd, sp, cp)
    sinv = jnp.where((ni & 2) == 0, sinv, -sinv)
    cosv = jnp.where(odd, cp, sp)
    cosv = jnp.where(((ni + 1) & 2) == 0, cosv, -cosv)
    emb = jnp.concatenate([sinv, cosv], axis=1)
    OUT = ytop_ref.shape[-1]
    y = jax.lax.dot_general(emb, w_ref[...], (((1,), (0,)), ((), ())),
                            preferred_element_type=jnp.float32)
    ytop_ref[...] = y[:, :OUT] + b_ref[...]
    part = jax.lax.dot_general(onehot, y[:, OUT:], (((0,), (0,)), ((), ())),
                               preferred_element_type=jnp.float32)

    @pl.when(i == 0)
    def _init():
        segacc_ref[...] = part

    @pl.when(i != 0)
    def _acc():
        segacc_ref[...] += part


def _pass_b(ytop_ref, lo_ref, hi_ref, segacc_ref, out_ref, *, T, B):
    i = pl.program_id(0)
    lo = lo_ref[...]
    hi = hi_ref[...]
    onehot = _onehot(i * T, lo_ref, hi_ref, T, B)
    inv_cnt = 1.0 / jnp.maximum((hi - lo).astype(jnp.float32), 1.0)
    ctx = jax.lax.dot_general(onehot * inv_cnt, segacc_ref[...],
                              (((1,), (0,)), ((), ())),
                              preferred_element_type=jnp.float32)
    out_ref[...] = ytop_ref[...] + ctx


def kernel(flat, cu_seqlens, W, b):
    n, d = flat.shape
    B = cu_seqlens.shape[0] - 1
    out_dim = W.shape[1]
    enc = W.shape[0] // (2 * d)
    half = enc // 2
    E = d * enc

    scales = XMIN * (XMAX / XMIN) ** (np.arange(half, dtype=np.float64)
                                      / max(half - 1, 1))
    Eh = E // 2
    s01 = np.zeros((d, Eh), np.float32)
    inv = np.zeros((Eh,), np.float32)
    for f in range(d):
        for j in range(half):
            s01[f, f * half + j] = 1.0
            inv[f * half + j] = 1.0 / scales[j]
    s2 = jnp.asarray(np.concatenate([s01, s01], axis=0)).astype(jnp.bfloat16)
    inv_row = jnp.asarray(inv).reshape(1, Eh)

    lo = cu_seqlens[:-1].reshape(1, B).astype(jnp.int32)
    hi = cu_seqlens[1:].reshape(1, B).astype(jnp.int32)
    b2 = b.reshape(1, out_dim)
    w2 = jnp.concatenate([W[:E, :], W[E:, :]], axis=1)
    sin_rows = np.array([f * enc + j for f in range(d) for j in range(half)])
    perm = np.concatenate([sin_rows, sin_rows + half])
    w2 = w2[perm, :]

    T = 2048
    K = n // T

    ytop, segacc = pl.pallas_call(
        functools.partial(_pass_a, T=T, E=E, B=B),
        grid=(K,),
        in_specs=[
            pl.BlockSpec((T, d), lambda i: (i, 0)),
            pl.BlockSpec((2 * d, E // 2), lambda i: (0, 0)),
            pl.BlockSpec((1, E // 2), lambda i: (0, 0)),
            pl.BlockSpec((1, B), lambda i: (0, 0)),
            pl.BlockSpec((1, B), lambda i: (0, 0)),
            pl.BlockSpec((E, 2 * out_dim), lambda i: (0, 0)),
            pl.BlockSpec((1, out_dim), lambda i: (0, 0)),
        ],
        out_specs=[
            pl.BlockSpec((T, out_dim), lambda i: (i, 0)),
            pl.BlockSpec((B, out_dim), lambda i: (0, 0)),
        ],
        out_shape=[
            jax.ShapeDtypeStruct((n, out_dim), jnp.float32),
            jax.ShapeDtypeStruct((B, out_dim), jnp.float32),
        ],
    )(flat, s2, inv_row, lo, hi, w2, b2)

    out = pl.pallas_call(
        functools.partial(_pass_b, T=T, B=B),
        grid=(K,),
        in_specs=[
            pl.BlockSpec((T, out_dim), lambda i: (i, 0)),
            pl.BlockSpec((1, B), lambda i: (0, 0)),
            pl.BlockSpec((1, B), lambda i: (0, 0)),
            pl.BlockSpec((B, out_dim), lambda i: (0, 0)),
        ],
        out_specs=pl.BlockSpec((T, out_dim), lambda i: (i, 0)),
        out_shape=jax.ShapeDtypeStruct((n, out_dim), jnp.float32),
    )(ytop, lo, hi, segacc)
    return out


# speedup vs baseline: 1.1177x; 1.1177x over previous
import functools

import jax
import jax.numpy as jnp
import numpy as np
from jax.experimental import pallas as pl
from jax.experimental.pallas import tpu as pltpu

XMIN = 0.1
XMAX = 2.0

_HI = jax.lax.Precision.HIGHEST

_PIO2_H1 = np.float32(1.5703125)
_PIO2_H2 = np.float32(np.pi / 2 - 1.5703125)
_PIO2_H3 = np.float32(np.pi / 2 - 1.5703125 - float(np.float32(np.pi / 2 - 1.5703125)))
_INV_PIO2 = np.float32(2.0 / np.pi)
_S1 = np.float32(-1.6666654611e-1)
_S2 = np.float32(8.3321608736e-3)
_S3 = np.float32(-1.9515295891e-4)
_C1 = np.float32(4.166664568298827e-2)
_C2 = np.float32(-1.388731625493765e-3)
_C3 = np.float32(2.443315711809948e-5)


def _fast_sin(ang):
    nf = jnp.floor(ang * _INV_PIO2 + 0.5)
    r = ang - nf * _PIO2_H1
    r = r - nf * _PIO2_H2
    r = r - nf * _PIO2_H3
    ni = nf.astype(jnp.int32)
    r2 = r * r
    sp = ((_S3 * r2 + _S2) * r2 + _S1) * (r2 * r) + r
    cp = ((_C3 * r2 + _C2) * r2 + _C1) * (r2 * r2) + (1.0 - 0.5 * r2)
    res = jnp.where((ni & 1) == 0, sp, cp)
    return jnp.where((ni & 2) == 0, res, -res)


def _onehot(base, lo_ref, hi_ref, T, B):
    idx = jax.lax.broadcasted_iota(jnp.int32, (T, B), 0) + base
    return jnp.where((idx >= lo_ref[...]) & (idx < hi_ref[...]), 1.0, 0.0)


def _pass_a(flat_ref, s2_ref, inv_ref, lo_ref, hi_ref, w_ref, b_ref,
            ytop_ref, segacc_ref, *, T, E, B):
    i = pl.program_id(0)
    onehot = _onehot(i * T, lo_ref, hi_ref, T, B)
    x = flat_ref[...]
    x1 = x.astype(jnp.bfloat16)
    x2 = (x - x1.astype(jnp.float32)).astype(jnp.bfloat16)
    xb = jax.lax.dot_general(jnp.concatenate([x1, x2], axis=1), s2_ref[...],
                             (((1,), (0,)), ((), ())),
                             preferred_element_type=jnp.float32)
    ang = xb * inv_ref[...]
    nf = jnp.floor(ang * _INV_PIO2 + 0.5)
    r = ang - nf * _PIO2_H1
    r = r - nf * _PIO2_H2
    r = r - nf * _PIO2_H3
    ni = nf.astype(jnp.int32)
    r2 = r * r
    sp = ((_S3 * r2 + _S2) * r2 + _S1) * (r2 * r) + r
    cp = ((_C3 * r2 + _C2) * r2 + _C1) * (r2 * r2) + (1.0 - 0.5 * r2)
    odd = (ni & 1) == 0
    sinv = jnp.where(odd, sp, cp)
    sinv = jnp.where((ni & 2) == 0, sinv, -sinv)
    cosv = jnp.where(odd, cp, sp)
    cosv = jnp.where(((ni + 1) & 2) == 0, cosv, -cosv)
    emb = jnp.concatenate([sinv, cosv], axis=1)
    OUT = ytop_ref.shape[-1]
    y = jax.lax.dot_general(emb, w_ref[...], (((1,), (0,)), ((), ())),
                            preferred_element_type=jnp.float32)
    ytop_ref[...] = y[:, :OUT] + b_ref[...]
    part = jax.lax.dot_general(onehot, y[:, OUT:], (((0,), (0,)), ((), ())),
                               preferred_element_type=jnp.float32)

    @pl.when(i == 0)
    def _init():
        segacc_ref[...] = part

    @pl.when(i != 0)
    def _acc():
        segacc_ref[...] += part


def _pass_b(ytop_ref, lo_ref, hi_ref, segacc_ref, out_ref, *, T, B):
    i = pl.program_id(0)
    lo = lo_ref[...]
    hi = hi_ref[...]
    onehot = _onehot(i * T, lo_ref, hi_ref, T, B)
    inv_cnt = 1.0 / jnp.maximum((hi - lo).astype(jnp.float32), 1.0)
    ctx = jax.lax.dot_general(onehot * inv_cnt, segacc_ref[...],
                              (((1,), (0,)), ((), ())),
                              preferred_element_type=jnp.float32)
    out_ref[...] = ytop_ref[...] + ctx


def kernel(flat, cu_seqlens, W, b):
    n, d = flat.shape
    B = cu_seqlens.shape[0] - 1
    out_dim = W.shape[1]
    enc = W.shape[0] // (2 * d)
    half = enc // 2
    E = d * enc

    scales = XMIN * (XMAX / XMIN) ** (np.arange(half, dtype=np.float64)
                                      / max(half - 1, 1))
    Eh = E // 2
    s01 = np.zeros((d, Eh), np.float32)
    inv = np.zeros((Eh,), np.float32)
    for f in range(d):
        for j in range(half):
            s01[f, f * half + j] = 1.0
            inv[f * half + j] = 1.0 / scales[j]
    s2 = jnp.asarray(np.concatenate([s01, s01], axis=0)).astype(jnp.bfloat16)
    inv_row = jnp.asarray(inv).reshape(1, Eh)

    lo = cu_seqlens[:-1].reshape(1, B).astype(jnp.int32)
    hi = cu_seqlens[1:].reshape(1, B).astype(jnp.int32)
    b2 = b.reshape(1, out_dim)
    w2 = jnp.concatenate([W[:E, :], W[E:, :]], axis=1)
    sin_rows = np.array([f * enc + j for f in range(d) for j in range(half)])
    perm = np.concatenate([sin_rows, sin_rows + half])
    w2 = w2[perm, :]

    T = 4096
    K = n // T
    TB = 8192
    KB = n // TB

    ytop, segacc = pl.pallas_call(
        functools.partial(_pass_a, T=T, E=E, B=B),
        grid=(K,),
        in_specs=[
            pl.BlockSpec((T, d), lambda i: (i, 0)),
            pl.BlockSpec((2 * d, E // 2), lambda i: (0, 0)),
            pl.BlockSpec((1, E // 2), lambda i: (0, 0)),
            pl.BlockSpec((1, B), lambda i: (0, 0)),
            pl.BlockSpec((1, B), lambda i: (0, 0)),
            pl.BlockSpec((E, 2 * out_dim), lambda i: (0, 0)),
            pl.BlockSpec((1, out_dim), lambda i: (0, 0)),
        ],
        out_specs=[
            pl.BlockSpec((T, out_dim), lambda i: (i, 0)),
            pl.BlockSpec((B, out_dim), lambda i: (0, 0)),
        ],
        out_shape=[
            jax.ShapeDtypeStruct((n, out_dim), jnp.float32),
            jax.ShapeDtypeStruct((B, out_dim), jnp.float32),
        ],
    )(flat, s2, inv_row, lo, hi, w2, b2)

    out = pl.pallas_call(
        functools.partial(_pass_b, T=TB, B=B),
        grid=(KB,),
        in_specs=[
            pl.BlockSpec((TB, out_dim), lambda i: (i, 0)),
            pl.BlockSpec((1, B), lambda i: (0, 0)),
            pl.BlockSpec((1, B), lambda i: (0, 0)),
            pl.BlockSpec((B, out_dim), lambda i: (0, 0)),
        ],
        out_specs=pl.BlockSpec((TB, out_dim), lambda i: (i, 0)),
        out_shape=jax.ShapeDtypeStruct((n, out_dim), jnp.float32),
    )(ytop, lo, hi, segacc)
    return out


# speedup vs baseline: 1.1423x; 1.0220x over previous
import functools

import jax
import jax.numpy as jnp
import numpy as np
from jax.experimental import pallas as pl
from jax.experimental.pallas import tpu as pltpu

XMIN = 0.1
XMAX = 2.0

_HI = jax.lax.Precision.HIGHEST

_PIO2_H1 = np.float32(1.5703125)
_PIO2_H2 = np.float32(np.pi / 2 - 1.5703125)
_PIO2_H3 = np.float32(np.pi / 2 - 1.5703125 - float(np.float32(np.pi / 2 - 1.5703125)))
_INV_PIO2 = np.float32(2.0 / np.pi)
_S1 = np.float32(-1.6666654611e-1)
_S2 = np.float32(8.3321608736e-3)
_S3 = np.float32(-1.9515295891e-4)
_C1 = np.float32(4.166664568298827e-2)
_C2 = np.float32(-1.388731625493765e-3)
_C3 = np.float32(2.443315711809948e-5)


def _fast_sin(ang):
    nf = jnp.floor(ang * _INV_PIO2 + 0.5)
    r = ang - nf * _PIO2_H1
    r = r - nf * _PIO2_H2
    r = r - nf * _PIO2_H3
    ni = nf.astype(jnp.int32)
    r2 = r * r
    sp = ((_S3 * r2 + _S2) * r2 + _S1) * (r2 * r) + r
    cp = ((_C3 * r2 + _C2) * r2 + _C1) * (r2 * r2) + (1.0 - 0.5 * r2)
    res = jnp.where((ni & 1) == 0, sp, cp)
    return jnp.where((ni & 2) == 0, res, -res)


def _onehot(base, lo_ref, hi_ref, T, B):
    idx = jax.lax.broadcasted_iota(jnp.int32, (T, B), 0) + base
    return jnp.where((idx >= lo_ref[...]) & (idx < hi_ref[...]), 1.0, 0.0)


def _fused(flat_ref, s2_ref, inv_ref, lo_ref, hi_ref, w_ref, b_ref,
           out_ref, ytop_ref, segacc_ref, *, T, K, E, B):
    i = pl.program_id(0)

    @pl.when(i < K)
    def _phase_a():
        _phase_a_body(i, flat_ref, s2_ref, inv_ref, lo_ref, hi_ref, w_ref,
                      b_ref, ytop_ref, segacc_ref, T=T, E=E, B=B)

    @pl.when(i >= K)
    def _phase_b():
        k = i - K
        onehot = _onehot(k * T, lo_ref, hi_ref, T, B)
        cnt = (hi_ref[...] - lo_ref[...]).astype(jnp.float32)
        inv_cnt = 1.0 / jnp.maximum(cnt, 1.0)
        ctx = jax.lax.dot_general(onehot * inv_cnt, segacc_ref[...],
                                  (((1,), (0,)), ((), ())),
                                  preferred_element_type=jnp.float32)
        out_ref[...] = ytop_ref[pl.ds(k * T, T), :] + ctx


def _phase_a_body(i, flat_ref, s2_ref, inv_ref, lo_ref, hi_ref, w_ref, b_ref,
                  ytop_ref, segacc_ref, *, T, E, B):
    onehot = _onehot(i * T, lo_ref, hi_ref, T, B)
    x = flat_ref[...]
    x1 = x.astype(jnp.bfloat16)
    x2 = (x - x1.astype(jnp.float32)).astype(jnp.bfloat16)
    xb = jax.lax.dot_general(jnp.concatenate([x1, x2], axis=1), s2_ref[...],
                             (((1,), (0,)), ((), ())),
                             preferred_element_type=jnp.float32)
    ang = xb * inv_ref[...]
    nf = jnp.floor(ang * _INV_PIO2 + 0.5)
    r = ang - nf * _PIO2_H1
    r = r - nf * _PIO2_H2
    r = r - nf * _PIO2_H3
    ni = nf.astype(jnp.int32)
    r2 = r * r
    sp = ((_S3 * r2 + _S2) * r2 + _S1) * (r2 * r) + r
    cp = ((_C3 * r2 + _C2) * r2 + _C1) * (r2 * r2) + (1.0 - 0.5 * r2)
    odd = (ni & 1) == 0
    sinv = jnp.where(odd, sp, cp)
    sinv = jnp.where((ni & 2) == 0, sinv, -sinv)
    cosv = jnp.where(odd, cp, sp)
    cosv = jnp.where(((ni + 1) & 2) == 0, cosv, -cosv)
    emb = jnp.concatenate([sinv, cosv], axis=1)
    OUT = ytop_ref.shape[-1]
    y = jax.lax.dot_general(emb, w_ref[...], (((1,), (0,)), ((), ())),
                            preferred_element_type=jnp.float32)
    ytop_ref[pl.ds(i * T, T), :] = y[:, :OUT] + b_ref[...]
    part = jax.lax.dot_general(onehot, y[:, OUT:], (((0,), (0,)), ((), ())),
                               preferred_element_type=jnp.float32)

    @pl.when(i == 0)
    def _init():
        segacc_ref[...] = part

    @pl.when(i != 0)
    def _acc():
        segacc_ref[...] += part


def kernel(flat, cu_seqlens, W, b):
    n, d = flat.shape
    B = cu_seqlens.shape[0] - 1
    out_dim = W.shape[1]
    enc = W.shape[0] // (2 * d)
    half = enc // 2
    E = d * enc

    scales = XMIN * (XMAX / XMIN) ** (np.arange(half, dtype=np.float64)
                                      / max(half - 1, 1))
    Eh = E // 2
    s01 = np.zeros((d, Eh), np.float32)
    inv = np.zeros((Eh,), np.float32)
    for f in range(d):
        for j in range(half):
            s01[f, f * half + j] = 1.0
            inv[f * half + j] = 1.0 / scales[j]
    s2 = jnp.asarray(np.concatenate([s01, s01], axis=0)).astype(jnp.bfloat16)
    inv_row = jnp.asarray(inv).reshape(1, Eh)

    lo = cu_seqlens[:-1].reshape(1, B).astype(jnp.int32)
    hi = cu_seqlens[1:].reshape(1, B).astype(jnp.int32)
    b2 = b.reshape(1, out_dim)
    w2 = jnp.concatenate([W[:E, :], W[E:, :]], axis=1)
    sin_rows = np.array([f * enc + j for f in range(d) for j in range(half)])
    perm = np.concatenate([sin_rows, sin_rows + half])
    w2 = w2[perm, :]

    T = 4096
    K = n // T

    out = pl.pallas_call(
        functools.partial(_fused, T=T, K=K, E=E, B=B),
        grid=(2 * K,),
        in_specs=[
            pl.BlockSpec((T, d), lambda i: (i % K, 0)),
            pl.BlockSpec((2 * d, E // 2), lambda i: (0, 0)),
            pl.BlockSpec((1, E // 2), lambda i: (0, 0)),
            pl.BlockSpec((1, B), lambda i: (0, 0)),
            pl.BlockSpec((1, B), lambda i: (0, 0)),
            pl.BlockSpec((E, 2 * out_dim), lambda i: (0, 0)),
            pl.BlockSpec((1, out_dim), lambda i: (0, 0)),
        ],
        out_specs=pl.BlockSpec((T, out_dim),
                               lambda i: (jnp.where(i < K, 0, i - K), 0)),
        out_shape=jax.ShapeDtypeStruct((n, out_dim), jnp.float32),
        scratch_shapes=[
            pltpu.VMEM((n, out_dim), jnp.float32),
            pltpu.VMEM((B, out_dim), jnp.float32),
        ],
    )(flat, s2, inv_row, lo, hi, w2, b2)
    return out


# speedup vs baseline: 1.1838x; 1.0364x over previous
import functools

import jax
import jax.numpy as jnp
import numpy as np
from jax.experimental import pallas as pl
from jax.experimental.pallas import tpu as pltpu

XMIN = 0.1
XMAX = 2.0

_HI = jax.lax.Precision.HIGHEST

_PIO2_H1 = np.float32(1.5703125)
_PIO2_H2 = np.float32(np.pi / 2 - 1.5703125)
_PIO2_H3 = np.float32(np.pi / 2 - 1.5703125 - float(np.float32(np.pi / 2 - 1.5703125)))
_INV_PIO2 = np.float32(2.0 / np.pi)
_S1 = np.float32(-1.6666654611e-1)
_S2 = np.float32(8.3321608736e-3)
_S3 = np.float32(-1.9515295891e-4)
_C1 = np.float32(4.166664568298827e-2)
_C2 = np.float32(-1.388731625493765e-3)
_C3 = np.float32(2.443315711809948e-5)


def _fast_sin(ang):
    nf = jnp.floor(ang * _INV_PIO2 + 0.5)
    r = ang - nf * _PIO2_H1
    r = r - nf * _PIO2_H2
    r = r - nf * _PIO2_H3
    ni = nf.astype(jnp.int32)
    r2 = r * r
    sp = ((_S3 * r2 + _S2) * r2 + _S1) * (r2 * r) + r
    cp = ((_C3 * r2 + _C2) * r2 + _C1) * (r2 * r2) + (1.0 - 0.5 * r2)
    res = jnp.where((ni & 1) == 0, sp, cp)
    return jnp.where((ni & 2) == 0, res, -res)


def _onehot(base, lo_ref, hi_ref, T, B):
    idx = jax.lax.broadcasted_iota(jnp.int32, (T, B), 0) + base
    return jnp.where((idx >= lo_ref[...]) & (idx < hi_ref[...]), 1.0, 0.0)


def _fused(flat_ref, s2_ref, inv_ref, lo_ref, hi_ref, w_ref, b_ref,
           out_ref, ytop_ref, segacc_ref, *, T, K, E, B):
    i = pl.program_id(0)

    @pl.when(i < K)
    def _phase_a():
        _phase_a_body(i, flat_ref, s2_ref, inv_ref, lo_ref, hi_ref, w_ref,
                      b_ref, ytop_ref, segacc_ref, T=T, E=E, B=B)

    @pl.when(i >= K)
    def _phase_b():
        k = i - K
        onehot = _onehot(k * T, lo_ref, hi_ref, T, B)
        cnt = (hi_ref[...] - lo_ref[...]).astype(jnp.float32)
        inv_cnt = 1.0 / jnp.maximum(cnt, 1.0)
        ctx = jax.lax.dot_general(onehot * inv_cnt, segacc_ref[...],
                                  (((1,), (0,)), ((), ())),
                                  preferred_element_type=jnp.float32)
        out_ref[...] = ytop_ref[pl.ds(k * T, T), :] + ctx


def _phase_a_body(i, flat_ref, s2_ref, inv_ref, lo_ref, hi_ref, w_ref, b_ref,
                  ytop_ref, segacc_ref, *, T, E, B):
    onehot = _onehot(i * T, lo_ref, hi_ref, T, B)
    x = flat_ref[...]
    x1 = x.astype(jnp.bfloat16)
    x2 = (x - x1.astype(jnp.float32)).astype(jnp.bfloat16)
    xb = jax.lax.dot_general(jnp.concatenate([x1, x2], axis=1), s2_ref[...],
                             (((1,), (0,)), ((), ())),
                             preferred_element_type=jnp.float32)
    ang = xb * inv_ref[...]
    nf = jnp.floor(ang * _INV_PIO2 + 0.5)
    r = ang - nf * _PIO2_H1
    r = r - nf * _PIO2_H2
    r = r - nf * _PIO2_H3
    ni = nf.astype(jnp.int32)
    r2 = r * r
    sp = ((_S3 * r2 + _S2) * r2 + _S1) * (r2 * r) + r
    cp = ((_C3 * r2 + _C2) * r2 + _C1) * (r2 * r2) + (1.0 - 0.5 * r2)
    odd = (ni & 1) == 0
    sinv = jnp.where(odd, sp, cp)
    sinv = jnp.where((ni & 2) == 0, sinv, -sinv)
    cosv = jnp.where(odd, cp, sp)
    cosv = jnp.where(((ni + 1) & 2) == 0, cosv, -cosv)
    emb = jnp.concatenate([sinv, cosv], axis=1)
    OUT = ytop_ref.shape[-1]
    y = jax.lax.dot_general(emb, w_ref[...], (((1,), (0,)), ((), ())),
                            preferred_element_type=jnp.float32)
    ytop_ref[pl.ds(i * T, T), :] = y[:, :OUT] + b_ref[...]
    part = jax.lax.dot_general(onehot, y[:, OUT:], (((0,), (0,)), ((), ())),
                               preferred_element_type=jnp.float32)

    @pl.when(i == 0)
    def _init():
        segacc_ref[...] = part

    @pl.when(i != 0)
    def _acc():
        segacc_ref[...] += part


def kernel(flat, cu_seqlens, W, b):
    n, d = flat.shape
    B = cu_seqlens.shape[0] - 1
    out_dim = W.shape[1]
    enc = W.shape[0] // (2 * d)
    half = enc // 2
    E = d * enc

    scales = XMIN * (XMAX / XMIN) ** (np.arange(half, dtype=np.float64)
                                      / max(half - 1, 1))
    Eh = E // 2
    s01 = np.zeros((d, Eh), np.float32)
    inv = np.zeros((Eh,), np.float32)
    for f in range(d):
        for j in range(half):
            s01[f, f * half + j] = 1.0
            inv[f * half + j] = 1.0 / scales[j]
    s2 = jnp.asarray(np.concatenate([s01, s01], axis=0)).astype(jnp.bfloat16)
    inv_row = jnp.asarray(inv).reshape(1, Eh)

    lo = cu_seqlens[:-1].reshape(1, B).astype(jnp.int32)
    hi = cu_seqlens[1:].reshape(1, B).astype(jnp.int32)
    b2 = b.reshape(1, out_dim)
    w2 = jnp.concatenate([W[:E, :], W[E:, :]], axis=1)
    sin_rows = np.array([f * enc + j for f in range(d) for j in range(half)])
    perm = np.concatenate([sin_rows, sin_rows + half])
    w2 = w2[perm, :]

    T = 8192
    K = n // T

    out = pl.pallas_call(
        functools.partial(_fused, T=T, K=K, E=E, B=B),
        grid=(2 * K,),
        in_specs=[
            pl.BlockSpec((T, d), lambda i: (i % K, 0)),
            pl.BlockSpec((2 * d, E // 2), lambda i: (0, 0)),
            pl.BlockSpec((1, E // 2), lambda i: (0, 0)),
            pl.BlockSpec((1, B), lambda i: (0, 0)),
            pl.BlockSpec((1, B), lambda i: (0, 0)),
            pl.BlockSpec((E, 2 * out_dim), lambda i: (0, 0)),
            pl.BlockSpec((1, out_dim), lambda i: (0, 0)),
        ],
        out_specs=pl.BlockSpec((T, out_dim),
                               lambda i: (jnp.where(i < K, 0, i - K), 0)),
        out_shape=jax.ShapeDtypeStruct((n, out_dim), jnp.float32),
        scratch_shapes=[
            pltpu.VMEM((n, out_dim), jnp.float32),
            pltpu.VMEM((B, out_dim), jnp.float32),
        ],
    )(flat, s2, inv_row, lo, hi, w2, b2)
    return out


# speedup vs baseline: 1.2231x; 1.0332x over previous
import functools

import jax
import jax.numpy as jnp
import numpy as np
from jax.experimental import pallas as pl
from jax.experimental.pallas import tpu as pltpu

XMIN = 0.1
XMAX = 2.0

_HI = jax.lax.Precision.HIGHEST

_PIO2_H1 = np.float32(1.5703125)
_PIO2_H2 = np.float32(np.pi / 2 - 1.5703125)
_INV_PIO2 = np.float32(2.0 / np.pi)
_S1 = np.float32(-0.1666664034128189)
_S2 = np.float32(0.008331716060638428)
_S3 = np.float32(-0.0001947511191247031)
_C1 = np.float32(-0.4999987781047821)
_C2 = np.float32(0.04165610671043396)
_C3 = np.float32(-0.0013598687946796417)


def _onehot(base, lo_ref, hi_ref, T, B):
    idx = jax.lax.broadcasted_iota(jnp.int32, (T, B), 0) + base
    return jnp.where((idx >= lo_ref[...]) & (idx < hi_ref[...]), 1.0, 0.0)


def _fused(flat_ref, s2_ref, inv_ref, lo_ref, hi_ref, w_ref, b_ref,
           out_ref, ytop_ref, segacc_ref, *, T, K, E, B):
    i = pl.program_id(0)

    @pl.when(i < K)
    def _phase_a():
        _phase_a_body(i, flat_ref, s2_ref, inv_ref, lo_ref, hi_ref, w_ref,
                      b_ref, ytop_ref, segacc_ref, T=T, E=E, B=B)

    @pl.when(i >= K)
    def _phase_b():
        k = i - K
        onehot = _onehot(k * T, lo_ref, hi_ref, T, B)
        cnt = (hi_ref[...] - lo_ref[...]).astype(jnp.float32)
        inv_cnt = 1.0 / jnp.maximum(cnt, 1.0)
        ctx = jax.lax.dot_general(onehot * inv_cnt, segacc_ref[...],
                                  (((1,), (0,)), ((), ())),
                                  preferred_element_type=jnp.float32)
        out_ref[...] = ytop_ref[pl.ds(k * T, T), :] + ctx


def _phase_a_body(i, flat_ref, s2_ref, inv_ref, lo_ref, hi_ref, w_ref, b_ref,
                  ytop_ref, segacc_ref, *, T, E, B):
    onehot = _onehot(i * T, lo_ref, hi_ref, T, B)
    x = flat_ref[...]
    x1 = x.astype(jnp.bfloat16)
    x2 = (x - x1.astype(jnp.float32)).astype(jnp.bfloat16)
    xb = jax.lax.dot_general(jnp.concatenate([x1, x2], axis=1), s2_ref[...],
                             (((1,), (0,)), ((), ())),
                             preferred_element_type=jnp.float32)
    ang = xb * inv_ref[...]
    nf = jnp.floor(ang * _INV_PIO2 + 0.5)
    r = ang - nf * _PIO2_H1
    r = r - nf * _PIO2_H2
    ni = nf.astype(jnp.int32)
    r2 = r * r
    sp = (((_S3 * r2 + _S2) * r2 + _S1) * r2 + 1.0) * r
    cp = ((_C3 * r2 + _C2) * r2 + _C1) * r2 + 1.0
    odd = (ni & 1) == 0
    sinv = jnp.where(odd, sp, cp)
    sinv = jnp.where((ni & 2) == 0, sinv, -sinv)
    cosv = jnp.where(odd, cp, sp)
    cosv = jnp.where(((ni + 1) & 2) == 0, cosv, -cosv)
    emb = jnp.concatenate([sinv, cosv], axis=1)
    OUT = ytop_ref.shape[-1]
    y = jax.lax.dot_general(emb, w_ref[...], (((1,), (0,)), ((), ())),
                            preferred_element_type=jnp.float32)
    ytop_ref[pl.ds(i * T, T), :] = y[:, :OUT] + b_ref[...]
    part = jax.lax.dot_general(onehot, y[:, OUT:], (((0,), (0,)), ((), ())),
                               preferred_element_type=jnp.float32)

    @pl.when(i == 0)
    def _init():
        segacc_ref[...] = part

    @pl.when(i != 0)
    def _acc():
        segacc_ref[...] += part


def kernel(flat, cu_seqlens, W, b):
    n, d = flat.shape
    B = cu_seqlens.shape[0] - 1
    out_dim = W.shape[1]
    enc = W.shape[0] // (2 * d)
    half = enc // 2
    E = d * enc

    scales = XMIN * (XMAX / XMIN) ** (np.arange(half, dtype=np.float64)
                                      / max(half - 1, 1))
    Eh = E // 2
    s01 = np.zeros((d, Eh), np.float32)
    inv = np.zeros((Eh,), np.float32)
    for f in range(d):
        for j in range(half):
            s01[f, f * half + j] = 1.0
            inv[f * half + j] = 1.0 / scales[j]
    s2 = jnp.asarray(np.concatenate([s01, s01], axis=0)).astype(jnp.bfloat16)
    inv_row = jnp.asarray(inv).reshape(1, Eh)

    lo = cu_seqlens[:-1].reshape(1, B).astype(jnp.int32)
    hi = cu_seqlens[1:].reshape(1, B).astype(jnp.int32)
    b2 = b.reshape(1, out_dim)
    w2 = jnp.concatenate([W[:E, :], W[E:, :]], axis=1)
    sin_rows = np.array([f * enc + j for f in range(d) for j in range(half)])
    perm = np.concatenate([sin_rows, sin_rows + half])
    w2 = w2[perm, :]

    T = 8192
    K = n // T

    out = pl.pallas_call(
        functools.partial(_fused, T=T, K=K, E=E, B=B),
        grid=(2 * K,),
        in_specs=[
            pl.BlockSpec((T, d), lambda i: (i % K, 0)),
            pl.BlockSpec((2 * d, E // 2), lambda i: (0, 0)),
            pl.BlockSpec((1, E // 2), lambda i: (0, 0)),
            pl.BlockSpec((1, B), lambda i: (0, 0)),
            pl.BlockSpec((1, B), lambda i: (0, 0)),
            pl.BlockSpec((E, 2 * out_dim), lambda i: (0, 0)),
            pl.BlockSpec((1, out_dim), lambda i: (0, 0)),
        ],
        out_specs=pl.BlockSpec((T, out_dim),
                               lambda i: (jnp.where(i < K, 0, i - K), 0)),
        out_shape=jax.ShapeDtypeStruct((n, out_dim), jnp.float32),
        scratch_shapes=[
            pltpu.VMEM((n, out_dim), jnp.float32),
            pltpu.VMEM((B, out_dim), jnp.float32),
        ],
    )(flat, s2, inv_row, lo, hi, w2, b2)
    return out


# speedup vs baseline: 1.2269x; 1.0031x over previous
import functools

import jax
import jax.numpy as jnp
import numpy as np
from jax.experimental import pallas as pl
from jax.experimental.pallas import tpu as pltpu

XMIN = 0.1
XMAX = 2.0

_PIO2_H1 = np.float32(1.5703125)
_PIO2_H2 = np.float32(np.pi / 2 - 1.5703125)
_INV_PIO2 = np.float32(2.0 / np.pi)
_S1 = np.float32(-0.1666664034128189)
_S2 = np.float32(0.008331716060638428)
_S3 = np.float32(-0.0001947511191247031)
_C1 = np.float32(-0.4999987781047821)
_C2 = np.float32(0.04165610671043396)
_C3 = np.float32(-0.0013598687946796417)


def _onehot(base, lo_ref, hi_ref, T, B):
    idx = jax.lax.broadcasted_iota(jnp.int32, (T, B), 0) + base
    return jnp.where((idx >= lo_ref[...]) & (idx < hi_ref[...]), 1.0, 0.0)


def _fused(flat_ref, s2_ref, inv_ref, lo_ref, hi_ref, w_ref, b_ref,
           out_ref, ytop_ref, segacc_ref, *, T, K, E, B):
    i = pl.program_id(0)

    @pl.when(i < K)
    def _phase_a():
        _phase_a_body(i, flat_ref, s2_ref, inv_ref, lo_ref, hi_ref, w_ref,
                      b_ref, ytop_ref, segacc_ref, T=T, E=E, B=B)

    @pl.when(i >= K)
    def _phase_b():
        k = i - K
        onehot = _onehot(k * T, lo_ref, hi_ref, T, B)
        cnt = (hi_ref[...] - lo_ref[...]).astype(jnp.float32)
        inv_cnt = 1.0 / jnp.maximum(cnt, 1.0)
        ctx = jax.lax.dot_general(onehot * inv_cnt, segacc_ref[...],
                                  (((1,), (0,)), ((), ())),
                                  preferred_element_type=jnp.float32)
        out_ref[...] = ytop_ref[pl.ds(k * T, T), :] + ctx


def _phase_a_body(i, flat_ref, s2_ref, inv_ref, lo_ref, hi_ref, w_ref, b_ref,
                  ytop_ref, segacc_ref, *, T, E, B):
    onehot = _onehot(i * T, lo_ref, hi_ref, T, B)
    x = flat_ref[...]
    x1 = x.astype(jnp.bfloat16)
    x2 = (x - x1.astype(jnp.float32)).astype(jnp.bfloat16)
    xb = jax.lax.dot_general(jnp.concatenate([x1, x2], axis=1), s2_ref[...],
                             (((1,), (0,)), ((), ())),
                             preferred_element_type=jnp.float32)
    ang = xb * inv_ref[...]
    nf = jnp.floor(ang * _INV_PIO2 + 0.5)
    r = ang - nf * _PIO2_H1
    r = r - nf * _PIO2_H2
    ni = nf.astype(jnp.int32)
    r2 = r * r
    sp = (((_S3 * r2 + _S2) * r2 + _S1) * r2 + 1.0) * r
    cp = ((_C3 * r2 + _C2) * r2 + _C1) * r2 + 1.0
    odd = (ni & 1) == 0
    sinv = jnp.where(odd, sp, cp)
    sinv = jnp.where((ni & 2) == 0, sinv, -sinv)
    cosv = jnp.where(odd, cp, sp)
    cosv = jnp.where(((ni + 1) & 2) == 0, cosv, -cosv)
    emb = jnp.concatenate([sinv, cosv], axis=1)
    OUT = ytop_ref.shape[-1]
    y = jax.lax.dot_general(emb, w_ref[...], (((1,), (0,)), ((), ())),
                            preferred_element_type=jnp.float32)
    ytop_ref[pl.ds(i * T, T), :] = y[:, :OUT] + b_ref[...]
    part = jax.lax.dot_general(onehot, y[:, OUT:], (((0,), (0,)), ((), ())),
                               preferred_element_type=jnp.float32)

    @pl.when(i == 0)
    def _init():
        segacc_ref[...] = part

    @pl.when(i != 0)
    def _acc():
        segacc_ref[...] += part


def kernel(flat, cu_seqlens, W, b):
    n, d = flat.shape
    B = cu_seqlens.shape[0] - 1
    out_dim = W.shape[1]
    enc = W.shape[0] // (2 * d)
    half = enc // 2
    E = d * enc

    scales = XMIN * (XMAX / XMIN) ** (np.arange(half, dtype=np.float64)
                                      / max(half - 1, 1))
    Eh = E // 2
    s01 = np.zeros((d, Eh), np.float32)
    inv = np.zeros((Eh,), np.float32)
    for f in range(d):
        for j in range(half):
            s01[f, f * half + j] = 1.0
            inv[f * half + j] = 1.0 / scales[j]
    s2 = jnp.asarray(np.concatenate([s01, s01], axis=0)).astype(jnp.bfloat16)
    inv_row = jnp.asarray(inv).reshape(1, Eh)

    lo = cu_seqlens[:-1].reshape(1, B).astype(jnp.int32)
    hi = cu_seqlens[1:].reshape(1, B).astype(jnp.int32)
    b2 = b.reshape(1, out_dim)
    w2 = jnp.concatenate([W[:E, :], W[E:, :]], axis=1)
    sin_rows = np.array([f * enc + j for f in range(d) for j in range(half)])
    perm = np.concatenate([sin_rows, sin_rows + half])
    w2 = w2[perm, :]

    T = 8192
    K = n // T

    out = pl.pallas_call(
        functools.partial(_fused, T=T, K=K, E=E, B=B),
        grid=(2 * K,),
        in_specs=[
            pl.BlockSpec((T, d), lambda i: (i % K, 0)),
            pl.BlockSpec((2 * d, E // 2), lambda i: (0, 0)),
            pl.BlockSpec((1, E // 2), lambda i: (0, 0)),
            pl.BlockSpec((1, B), lambda i: (0, 0)),
            pl.BlockSpec((1, B), lambda i: (0, 0)),
            pl.BlockSpec((E, 2 * out_dim), lambda i: (0, 0)),
            pl.BlockSpec((1, out_dim), lambda i: (0, 0)),
        ],
        out_specs=pl.BlockSpec((T, out_dim),
                               lambda i: (jnp.where(i < K, 0, i - K), 0)),
        out_shape=jax.ShapeDtypeStruct((n, out_dim), jnp.float32),
        scratch_shapes=[
            pltpu.VMEM((n, out_dim), jnp.float32),
            pltpu.VMEM((B, out_dim), jnp.float32),
        ],
    )(flat, s2, inv_row, lo, hi, w2, b2)
    return out
